# baseline (speedup 1.0000x reference)
import functools

import jax
import jax.numpy as jnp
from jax import lax
from jax.experimental import pallas as pl
from jax.experimental.pallas import tpu as pltpu
from jax.experimental.pallas import tpu_sc as plsc

N = 10000
E = 320000
DF = 128
HF = 64
HID = 64
LAT = 32
OUTD = 64
MAXLS = 10.0

NC, NS = 2, 16
NW = NC * NS
EPW = 10240
E_PAD = NW * EPW
CB = 128
NCH = EPW // CB
RPT = 640
R_ACC = NS * RPT
PAD_DST = 10008
DEG_ACC = 10240

BN = 1024
GRID = (N + BN - 1) // BN


def _sc_mesh():
    return plsc.VectorSubcoreMesh(core_axis_name="c", subcore_axis_name="s")


EPD = E // NW


@functools.partial(
    pl.kernel,
    mesh=_sc_mesh(),
    out_type=jax.ShapeDtypeStruct((NW, DEG_ACC), jnp.float32),
    scratch_types=[
        pltpu.VMEM((EPD,), jnp.int32),
        pltpu.VMEM((DEG_ACC,), jnp.float32),
    ],
    compiler_params=pltpu.CompilerParams(needs_layout_passes=False),
)
def _sc_degree(dst_hbm, out_hbm, dst_v, acc_v):
    cid = lax.axis_index("c")
    sid = lax.axis_index("s")
    wid = cid * NS + sid
    pltpu.sync_copy(dst_hbm.at[pl.ds(wid * EPD, EPD)], dst_v)

    def zero(i, _):
        acc_v[pl.ds(i * 16, 16)] = jnp.zeros((16,), jnp.float32)
        return 0

    lax.fori_loop(0, DEG_ACC // 16, zero, 0)

    ones = jnp.ones((16,), jnp.float32)

    def body(i, _):
        idx = dst_v[pl.ds(i * 16, 16)]
        plsc.addupdate_scatter(acc_v, [idx], ones)
        return 0

    lax.fori_loop(0, EPD // 16, body, 0)
    pltpu.sync_copy(acc_v, out_hbm.at[wid])


FAST_CID = 0
FAST_NCH = 106
SLOW_NCH = 51
ECH = E // CB
FROWS = 16 * FAST_NCH
SROWS = 16 * SLOW_NCH
TAIL_REAL = ECH - FROWS - 15 * SLOW_NCH
TAIL_PAD = SLOW_NCH - TAIL_REAL


@functools.partial(
    pl.kernel,
    mesh=_sc_mesh(),
    out_type=jax.ShapeDtypeStruct((NC, R_ACC, HF), jnp.float32),
    scratch_types=[
        pltpu.VMEM((FAST_NCH, CB), jnp.int32),
        pltpu.VMEM((FAST_NCH, CB), jnp.int32),
        pltpu.VMEM((4, CB, HF), jnp.float32),
        pltpu.VMEM_SHARED((R_ACC, HF), jnp.float32),
        pltpu.SemaphoreType.DMA((4,)),
        pltpu.SemaphoreType.DMA((4,)),
    ],
    compiler_params=pltpu.CompilerParams(use_tc_tiling_on_sc=False),
)
def _sc_agg(m_hbm, src3, dst3, out_hbm,
            src_v, dst_v, rows_v, acc_s, gsem, ssem):
    cid = lax.axis_index("c")
    sid = lax.axis_index("s")
    is_fast = cid == FAST_CID
    nch = jnp.where(is_fast, FAST_NCH, SLOW_NCH)

    @pl.when(is_fast)
    def _():
        pltpu.sync_copy(src3.at[pl.ds(sid * FAST_NCH, FAST_NCH)], src_v)
        pltpu.sync_copy(dst3.at[pl.ds(sid * FAST_NCH, FAST_NCH)], dst_v)

    is_tail = jnp.logical_and(jnp.logical_not(is_fast), sid == NS - 1)

    @pl.when(jnp.logical_and(jnp.logical_not(is_fast), sid < NS - 1))
    def _():
        pltpu.sync_copy(src3.at[pl.ds(FROWS + sid * SLOW_NCH, SLOW_NCH)],
                        src_v.at[pl.ds(0, SLOW_NCH)])
        pltpu.sync_copy(dst3.at[pl.ds(FROWS + sid * SLOW_NCH, SLOW_NCH)],
                        dst_v.at[pl.ds(0, SLOW_NCH)])

    @pl.when(is_tail)
    def _():
        base = FROWS + (NS - 1) * SLOW_NCH
        pltpu.sync_copy(src3.at[pl.ds(base, TAIL_REAL)],
                        src_v.at[pl.ds(0, TAIL_REAL)])
        pltpu.sync_copy(dst3.at[pl.ds(base, TAIL_REAL)],
                        dst_v.at[pl.ds(0, TAIL_REAL)])

        def padfill(i, _):
            r = TAIL_REAL + i // (CB // 16)
            c = (i % (CB // 16)) * 16
            src_v[r, pl.ds(c, 16)] = jnp.zeros((16,), jnp.int32)
            dst_v[r, pl.ds(c, 16)] = jnp.full((16,), PAD_DST, jnp.int32)
            return 0

        lax.fori_loop(0, TAIL_PAD * (CB // 16), padfill, 0)

    def zstore(i, _):
        rows_v[0, i // 4, pl.ds((i % 4) * 16, 16)] = jnp.zeros(
            (16,), jnp.float32)
        return 0

    lax.fori_loop(0, CB * 4, zstore, 0)
    for t in range(RPT // CB):
        pltpu.sync_copy(rows_v.at[0],
                        acc_s.at[pl.ds(sid * RPT + t * CB, CB)])
    plsc.subcore_barrier()

    pltpu.async_copy(m_hbm.at[src_v.at[0]], rows_v.at[0], gsem.at[0])
    pltpu.async_copy(m_hbm.at[src_v.at[1]], rows_v.at[1], gsem.at[1])

    def body(j, _):
        b = lax.rem(j, 4)
        pltpu.make_async_copy(m_hbm.at[src_v.at[j]], rows_v.at[b],
                              gsem.at[b]).wait()

        @pl.when(j >= 2)
        def _():
            b2 = lax.rem(j - 2, 4)
            pltpu.make_async_copy(rows_v.at[b2],
                                  acc_s.at[dst_v.at[j - 2]],
                                  ssem.at[b2]).wait()

        @pl.when(j + 2 < nch)
        def _():
            b3 = lax.rem(j + 2, 4)
            pltpu.async_copy(m_hbm.at[src_v.at[j + 2]], rows_v.at[b3],
                             gsem.at[b3])

        pltpu.async_copy(rows_v.at[b], acc_s.at[dst_v.at[j]],
                         ssem.at[b], add=True)
        return 0

    lax.fori_loop(0, nch, body, 0)
    for k in (2, 1):
        jj = nch - k
        b = lax.rem(jj, 4)
        pltpu.make_async_copy(rows_v.at[b], acc_s.at[dst_v.at[jj]],
                              ssem.at[b]).wait()
    plsc.subcore_barrier()
    pltpu.sync_copy(acc_s.at[pl.ds(sid * RPT, RPT)],
                    out_hbm.at[cid, pl.ds(sid * RPT, RPT)])


def _dense1a_body(x, w1, u_o):
    u_o[...] = jnp.dot(x[...], w1[...], preferred_element_type=jnp.float32)


def _dense1b_body(degp, u, lo_o, hi_o, dsc_o):
    deg = jnp.maximum(jnp.sum(degp[...], axis=0), 1.0)
    dsc = lax.rsqrt(deg)
    m = u[...] * dsc[:, None]
    lo_o[...] = m[:, :HF]
    hi_o[...] = m[:, HF:]
    dsc_o[...] = dsc


def _dense2h_body(a, dsc, w, m_o):
    h = jnp.maximum((a[0] + a[1]) * dsc[...][:, None], 0.0)
    m_o[...] = jnp.dot(h, w[...],
                       preferred_element_type=jnp.float32) * dsc[...][:, None]


def _dense3a_body(a, dsc, eps, wo, y_o):
    t = (a[0] + a[1]) * dsc[...][:, None]
    z1 = t[:, 0:LAT] + eps[...] * jnp.exp(jnp.minimum(t[:, LAT:], MAXLS))
    y_o[...] = jnp.dot(z1, wo[...], preferred_element_type=jnp.float32)


def _dense3b_body(a, dsc, eps, wo, bo, y1, z_o):
    t = (a[0] + a[1]) * dsc[...][:, None]
    z2 = t[:, 0:LAT] + eps[...] * jnp.exp(jnp.minimum(t[:, LAT:], MAXLS))
    z_o[...] = (jnp.dot(z2, wo[...], preferred_element_type=jnp.float32)
                + y1[...] + bo[...])


_dense1a = pl.pallas_call(
    _dense1a_body,
    grid=(GRID,),
    in_specs=[
        pl.BlockSpec((BN, DF), lambda i: (i, 0)),
        pl.BlockSpec((DF, DF), lambda i: (0, 0)),
    ],
    out_specs=pl.BlockSpec((BN, DF), lambda i: (i, 0)),
    out_shape=jax.ShapeDtypeStruct((N, DF), jnp.float32),
)

_dense1b = pl.pallas_call(
    _dense1b_body,
    grid=(GRID,),
    in_specs=[
        pl.BlockSpec((NW, BN), lambda i: (0, i)),
        pl.BlockSpec((BN, DF), lambda i: (i, 0)),
    ],
    out_specs=[
        pl.BlockSpec((BN, HF), lambda i: (i, 0)),
        pl.BlockSpec((BN, HF), lambda i: (i, 0)),
        pl.BlockSpec((BN,), lambda i: (i,)),
    ],
    out_shape=[
        jax.ShapeDtypeStruct((N, HF), jnp.float32),
        jax.ShapeDtypeStruct((N, HF), jnp.float32),
        jax.ShapeDtypeStruct((N,), jnp.float32),
    ],
)

_agg_spec = pl.BlockSpec((NC, BN, HF), lambda i: (0, i, 0))

_dense2h = pl.pallas_call(
    _dense2h_body,
    grid=(GRID,),
    in_specs=[
        _agg_spec,
        pl.BlockSpec((BN,), lambda i: (i,)),
        pl.BlockSpec((HF, HF), lambda i: (0, 0)),
    ],
    out_specs=pl.BlockSpec((BN, HF), lambda i: (i, 0)),
    out_shape=jax.ShapeDtypeStruct((N, HF), jnp.float32),
)

_dense3a = pl.pallas_call(
    _dense3a_body,
    grid=(GRID,),
    in_specs=[
        _agg_spec,
        pl.BlockSpec((BN,), lambda i: (i,)),
        pl.BlockSpec((BN, LAT), lambda i: (i, 0)),
        pl.BlockSpec((LAT, OUTD), lambda i: (0, 0)),
    ],
    out_specs=pl.BlockSpec((BN, OUTD), lambda i: (i, 0)),
    out_shape=jax.ShapeDtypeStruct((N, OUTD), jnp.float32),
)

_dense3b = pl.pallas_call(
    _dense3b_body,
    grid=(GRID,),
    in_specs=[
        _agg_spec,
        pl.BlockSpec((BN,), lambda i: (i,)),
        pl.BlockSpec((BN, LAT), lambda i: (i, 0)),
        pl.BlockSpec((LAT, OUTD), lambda i: (0, 0)),
        pl.BlockSpec((1, OUTD), lambda i: (0, 0)),
        pl.BlockSpec((BN, OUTD), lambda i: (i, 0)),
    ],
    out_specs=pl.BlockSpec((BN, OUTD), lambda i: (i, 0)),
    out_shape=jax.ShapeDtypeStruct((N, OUTD), jnp.float32),
)


def kernel(x, W1_a, Wmu_a, Wls_a, W1_b, Wmu_b, Wls_b, W_out, b_out,
           edge_index):
    f32 = jnp.float32
    W1c = jnp.concatenate([W1_a, W1_b], axis=1)
    W2A = jnp.concatenate([Wmu_a, Wls_a], axis=1)
    W2B = jnp.concatenate([Wmu_b, Wls_b], axis=1)
    ke_a, ke_b = jax.random.split(jax.random.key(42), 2)

    src3 = edge_index[0].reshape(ECH, CB)
    dst3 = edge_index[1].reshape(ECH, CB)

    degp = _sc_degree(edge_index[1])
    ka, kb, _ = lax.optimization_barrier((ke_a, ke_b, degp))
    eps_a = jax.random.normal(ka, (N, LAT), dtype=f32)
    eps_b = jax.random.normal(kb, (N, LAT), dtype=f32)
    u = _dense1a(x, W1c)
    m1lo, m1hi, dsc = _dense1b(degp, u)
    a1lo = _sc_agg(m1lo, src3, dst3)
    m2lo = _dense2h(a1lo, dsc, W2A)
    a1hi = _sc_agg(m1hi, src3, dst3)
    m2hi = _dense2h(a1hi, dsc, W2B)
    a2lo = _sc_agg(m2lo, src3, dst3)
    a2hi = _sc_agg(m2hi, src3, dst3)
    y1 = _dense3a(a2lo, dsc, eps_a, W_out[:LAT])
    a2hi, y1 = lax.optimization_barrier((a2hi, y1))
    z = _dense3b(a2hi, dsc, eps_b, W_out[LAT:], b_out.reshape(1, OUTD), y1)
    return z


# speedup vs baseline: 1.0353x; 1.0353x over previous
import functools

import jax
import jax.numpy as jnp
from jax import lax
from jax.experimental import pallas as pl
from jax.experimental.pallas import tpu as pltpu
from jax.experimental.pallas import tpu_sc as plsc

N = 10000
E = 320000
DF = 128
HF = 64
HID = 64
LAT = 32
OUTD = 64
MAXLS = 10.0

NC, NS = 2, 16
NW = NC * NS
EPW = 10240
E_PAD = NW * EPW
CB = 128
NCH = EPW // CB
RPT = 640
R_ACC = NS * RPT
PAD_DST = 10008
DEG_ACC = 10240

BN = 1024
GRID = (N + BN - 1) // BN


def _sc_mesh():
    return plsc.VectorSubcoreMesh(core_axis_name="c", subcore_axis_name="s")


EPD = E // NW


@functools.partial(
    pl.kernel,
    mesh=_sc_mesh(),
    out_type=jax.ShapeDtypeStruct((NW, DEG_ACC), jnp.float32),
    scratch_types=[
        pltpu.VMEM((EPD,), jnp.int32),
        pltpu.VMEM((DEG_ACC,), jnp.float32),
    ],
    compiler_params=pltpu.CompilerParams(needs_layout_passes=False),
)
def _sc_degree(dst_hbm, out_hbm, dst_v, acc_v):
    cid = lax.axis_index("c")
    sid = lax.axis_index("s")
    wid = cid * NS + sid
    pltpu.sync_copy(dst_hbm.at[pl.ds(wid * EPD, EPD)], dst_v)

    def zero(i, _):
        acc_v[pl.ds(i * 16, 16)] = jnp.zeros((16,), jnp.float32)
        return 0

    lax.fori_loop(0, DEG_ACC // 16, zero, 0)

    ones = jnp.ones((16,), jnp.float32)

    def body(i, _):
        idx = dst_v[pl.ds(i * 16, 16)]
        plsc.addupdate_scatter(acc_v, [idx], ones)
        return 0

    lax.fori_loop(0, EPD // 16, body, 0)
    pltpu.sync_copy(acc_v, out_hbm.at[wid])


FAST_CID = 0
FAST_NCH = 106
SLOW_NCH = 51
ECH = E // CB
FROWS = 16 * FAST_NCH
SROWS = 16 * SLOW_NCH
TAIL_REAL = ECH - FROWS - 15 * SLOW_NCH
TAIL_PAD = SLOW_NCH - TAIL_REAL


@functools.partial(
    pl.kernel,
    mesh=_sc_mesh(),
    out_type=jax.ShapeDtypeStruct((NC, R_ACC, HF), jnp.float32),
    scratch_types=[
        pltpu.VMEM((FAST_NCH, CB), jnp.int32),
        pltpu.VMEM((FAST_NCH, CB), jnp.int32),
        pltpu.VMEM((4, CB, HF), jnp.float32),
        pltpu.VMEM_SHARED((R_ACC, HF), jnp.float32),
        pltpu.SemaphoreType.DMA((4,)),
        pltpu.SemaphoreType.DMA((4,)),
    ],
    compiler_params=pltpu.CompilerParams(use_tc_tiling_on_sc=False),
)
def _sc_agg(m_hbm, src3, dst3, out_hbm,
            src_v, dst_v, rows_v, acc_s, gsem, ssem):
    cid = lax.axis_index("c")
    sid = lax.axis_index("s")
    is_fast = cid == FAST_CID
    nch = jnp.where(is_fast, FAST_NCH, SLOW_NCH)

    @pl.when(is_fast)
    def _():
        pltpu.sync_copy(src3.at[pl.ds(sid * FAST_NCH, FAST_NCH)], src_v)
        pltpu.sync_copy(dst3.at[pl.ds(sid * FAST_NCH, FAST_NCH)], dst_v)

    is_tail = jnp.logical_and(jnp.logical_not(is_fast), sid == NS - 1)

    @pl.when(jnp.logical_and(jnp.logical_not(is_fast), sid < NS - 1))
    def _():
        pltpu.sync_copy(src3.at[pl.ds(FROWS + sid * SLOW_NCH, SLOW_NCH)],
                        src_v.at[pl.ds(0, SLOW_NCH)])
        pltpu.sync_copy(dst3.at[pl.ds(FROWS + sid * SLOW_NCH, SLOW_NCH)],
                        dst_v.at[pl.ds(0, SLOW_NCH)])

    @pl.when(is_tail)
    def _():
        base = FROWS + (NS - 1) * SLOW_NCH
        pltpu.sync_copy(src3.at[pl.ds(base, TAIL_REAL)],
                        src_v.at[pl.ds(0, TAIL_REAL)])
        pltpu.sync_copy(dst3.at[pl.ds(base, TAIL_REAL)],
                        dst_v.at[pl.ds(0, TAIL_REAL)])

        def padfill(i, _):
            r = TAIL_REAL + i // (CB // 16)
            c = (i % (CB // 16)) * 16
            src_v[r, pl.ds(c, 16)] = jnp.zeros((16,), jnp.int32)
            dst_v[r, pl.ds(c, 16)] = jnp.full((16,), PAD_DST, jnp.int32)
            return 0

        lax.fori_loop(0, TAIL_PAD * (CB // 16), padfill, 0)

    def zstore(i, _):
        rows_v[0, i // 4, pl.ds((i % 4) * 16, 16)] = jnp.zeros(
            (16,), jnp.float32)
        return 0

    lax.fori_loop(0, CB * 4, zstore, 0)
    for t in range(RPT // CB):
        pltpu.sync_copy(rows_v.at[0],
                        acc_s.at[pl.ds(sid * RPT + t * CB, CB)])
    plsc.subcore_barrier()

    pltpu.async_copy(m_hbm.at[src_v.at[0]], rows_v.at[0], gsem.at[0])
    pltpu.async_copy(m_hbm.at[src_v.at[1]], rows_v.at[1], gsem.at[1])

    def body(j, _):
        b = lax.rem(j, 4)
        pltpu.make_async_copy(m_hbm.at[src_v.at[j]], rows_v.at[b],
                              gsem.at[b]).wait()

        @pl.when(j >= 2)
        def _():
            b2 = lax.rem(j - 2, 4)
            pltpu.make_async_copy(rows_v.at[b2],
                                  acc_s.at[dst_v.at[j - 2]],
                                  ssem.at[b2]).wait()

        @pl.when(j + 2 < nch)
        def _():
            b3 = lax.rem(j + 2, 4)
            pltpu.async_copy(m_hbm.at[src_v.at[j + 2]], rows_v.at[b3],
                             gsem.at[b3])

        pltpu.async_copy(rows_v.at[b], acc_s.at[dst_v.at[j]],
                         ssem.at[b], add=True)
        return 0

    lax.fori_loop(0, nch, body, 0)
    for k in (2, 1):
        jj = nch - k
        b = lax.rem(jj, 4)
        pltpu.make_async_copy(rows_v.at[b], acc_s.at[dst_v.at[jj]],
                              ssem.at[b]).wait()
    plsc.subcore_barrier()
    pltpu.sync_copy(acc_s.at[pl.ds(sid * RPT, RPT)],
                    out_hbm.at[cid, pl.ds(sid * RPT, RPT)])


def _dense1a_body(x, w1, u_o):
    u_o[...] = jnp.dot(x[...], w1[...], preferred_element_type=jnp.float32)


def _dense1b_body(degp, u, lo_o, hi_o, dsc_o):
    deg = jnp.maximum(jnp.sum(degp[...], axis=0), 1.0)
    dsc = lax.rsqrt(deg)
    m = u[...] * dsc[:, None]
    lo_o[...] = m[:, :HF]
    hi_o[...] = m[:, HF:]
    dsc_o[...] = dsc


def _dense2h_body(a, dsc, w, m_o):
    h = jnp.maximum((a[0] + a[1]) * dsc[...][:, None], 0.0)
    m_o[...] = jnp.dot(h, w[...],
                       preferred_element_type=jnp.float32) * dsc[...][:, None]


def _dense3_body(alo, ahi, dsc, eps, wo, bo, z_o):
    t = jnp.concatenate([alo[0] + alo[1], ahi[0] + ahi[1]],
                        axis=1) * dsc[...][:, None]
    mu = jnp.concatenate([t[:, 0:LAT], t[:, 2 * LAT:3 * LAT]], axis=1)
    ls = jnp.concatenate([t[:, LAT:2 * LAT], t[:, 3 * LAT:4 * LAT]], axis=1)
    z = mu + eps[...] * jnp.exp(jnp.minimum(ls, MAXLS))
    z_o[...] = jnp.dot(z, wo[...],
                       preferred_element_type=jnp.float32) + bo[...]


_dense1a = pl.pallas_call(
    _dense1a_body,
    grid=(GRID,),
    in_specs=[
        pl.BlockSpec((BN, DF), lambda i: (i, 0)),
        pl.BlockSpec((DF, DF), lambda i: (0, 0)),
    ],
    out_specs=pl.BlockSpec((BN, DF), lambda i: (i, 0)),
    out_shape=jax.ShapeDtypeStruct((N, DF), jnp.float32),
)

_dense1b = pl.pallas_call(
    _dense1b_body,
    grid=(GRID,),
    in_specs=[
        pl.BlockSpec((NW, BN), lambda i: (0, i)),
        pl.BlockSpec((BN, DF), lambda i: (i, 0)),
    ],
    out_specs=[
        pl.BlockSpec((BN, HF), lambda i: (i, 0)),
        pl.BlockSpec((BN, HF), lambda i: (i, 0)),
        pl.BlockSpec((BN,), lambda i: (i,)),
    ],
    out_shape=[
        jax.ShapeDtypeStruct((N, HF), jnp.float32),
        jax.ShapeDtypeStruct((N, HF), jnp.float32),
        jax.ShapeDtypeStruct((N,), jnp.float32),
    ],
)

_agg_spec = pl.BlockSpec((NC, BN, HF), lambda i: (0, i, 0))

_dense2h = pl.pallas_call(
    _dense2h_body,
    grid=(GRID,),
    in_specs=[
        _agg_spec,
        pl.BlockSpec((BN,), lambda i: (i,)),
        pl.BlockSpec((HF, HF), lambda i: (0, 0)),
    ],
    out_specs=pl.BlockSpec((BN, HF), lambda i: (i, 0)),
    out_shape=jax.ShapeDtypeStruct((N, HF), jnp.float32),
)

_dense3 = pl.pallas_call(
    _dense3_body,
    grid=(GRID,),
    in_specs=[
        _agg_spec,
        _agg_spec,
        pl.BlockSpec((BN,), lambda i: (i,)),
        pl.BlockSpec((BN, 2 * LAT), lambda i: (i, 0)),
        pl.BlockSpec((2 * LAT, OUTD), lambda i: (0, 0)),
        pl.BlockSpec((1, OUTD), lambda i: (0, 0)),
    ],
    out_specs=pl.BlockSpec((BN, OUTD), lambda i: (i, 0)),
    out_shape=jax.ShapeDtypeStruct((N, OUTD), jnp.float32),
)


def kernel(x, W1_a, Wmu_a, Wls_a, W1_b, Wmu_b, Wls_b, W_out, b_out,
           edge_index):
    f32 = jnp.float32
    W1c = jnp.concatenate([W1_a, W1_b], axis=1)
    W2A = jnp.concatenate([Wmu_a, Wls_a], axis=1)
    W2B = jnp.concatenate([Wmu_b, Wls_b], axis=1)
    ke_a, ke_b = jax.random.split(jax.random.key(42), 2)

    src3 = edge_index[0].reshape(ECH, CB)
    dst3 = edge_index[1].reshape(ECH, CB)

    eps_a = jax.random.normal(ke_a, (N, LAT), dtype=f32)
    eps_b = jax.random.normal(ke_b, (N, LAT), dtype=f32)
    eps = jnp.concatenate([eps_a, eps_b], axis=1)

    degp = _sc_degree(edge_index[1])
    u = _dense1a(x, W1c)
    m1lo, m1hi, dsc = _dense1b(degp, u)
    a1lo = _sc_agg(m1lo, src3, dst3)
    m2lo = _dense2h(a1lo, dsc, W2A)
    a1hi = _sc_agg(m1hi, src3, dst3)
    m2hi = _dense2h(a1hi, dsc, W2B)
    a2lo = _sc_agg(m2lo, src3, dst3)
    a2hi = _sc_agg(m2hi, src3, dst3)
    z = _dense3(a2lo, a2hi, dsc, eps, W_out, b_out.reshape(1, OUTD))
    return z


# speedup vs baseline: 1.0418x; 1.0063x over previous
import functools

import jax
import jax.numpy as jnp
from jax import lax
from jax.experimental import pallas as pl
from jax.experimental.pallas import tpu as pltpu
from jax.experimental.pallas import tpu_sc as plsc

N = 10000
E = 320000
DF = 128
HF = 64
HID = 64
LAT = 32
OUTD = 64
MAXLS = 10.0

NC, NS = 2, 16
NW = NC * NS
EPW = 10240
E_PAD = NW * EPW
CB = 128
NCH = EPW // CB
RPT = 640
R_ACC = NS * RPT
PAD_DST = 10008
DEG_ACC = 10240

BN = 1024
GRID = (N + BN - 1) // BN


def _sc_mesh():
    return plsc.VectorSubcoreMesh(core_axis_name="c", subcore_axis_name="s")


EPD = E // NW


@functools.partial(
    pl.kernel,
    mesh=_sc_mesh(),
    out_type=jax.ShapeDtypeStruct((NW, DEG_ACC), jnp.float32),
    scratch_types=[
        pltpu.VMEM((EPD,), jnp.int32),
        pltpu.VMEM((DEG_ACC,), jnp.float32),
    ],
    compiler_params=pltpu.CompilerParams(needs_layout_passes=False),
)
def _sc_degree(dst_hbm, out_hbm, dst_v, acc_v):
    cid = lax.axis_index("c")
    sid = lax.axis_index("s")
    wid = cid * NS + sid
    pltpu.sync_copy(dst_hbm.at[pl.ds(wid * EPD, EPD)], dst_v)

    def zero(i, _):
        acc_v[pl.ds(i * 16, 16)] = jnp.zeros((16,), jnp.float32)
        return 0

    lax.fori_loop(0, DEG_ACC // 16, zero, 0)

    ones = jnp.ones((16,), jnp.float32)

    def body(i, _):
        idx = dst_v[pl.ds(i * 16, 16)]
        plsc.addupdate_scatter(acc_v, [idx], ones)
        return 0

    lax.fori_loop(0, EPD // 16, body, 0)
    pltpu.sync_copy(acc_v, out_hbm.at[wid])


FAST_CID = 0
FAST_NCH = 105
SLOW_NCH = 52
ECH = E // CB
FROWS = 16 * FAST_NCH
SROWS = 16 * SLOW_NCH
TAIL_REAL = ECH - FROWS - 15 * SLOW_NCH
TAIL_PAD = SLOW_NCH - TAIL_REAL


@functools.partial(
    pl.kernel,
    mesh=_sc_mesh(),
    out_type=jax.ShapeDtypeStruct((NC, R_ACC, HF), jnp.float32),
    scratch_types=[
        pltpu.VMEM((FAST_NCH, CB), jnp.int32),
        pltpu.VMEM((FAST_NCH, CB), jnp.int32),
        pltpu.VMEM((4, CB, HF), jnp.float32),
        pltpu.VMEM_SHARED((R_ACC, HF), jnp.float32),
        pltpu.SemaphoreType.DMA((4,)),
        pltpu.SemaphoreType.DMA((4,)),
    ],
    compiler_params=pltpu.CompilerParams(use_tc_tiling_on_sc=False),
)
def _sc_agg(m_hbm, src3, dst3, out_hbm,
            src_v, dst_v, rows_v, acc_s, gsem, ssem):
    cid = lax.axis_index("c")
    sid = lax.axis_index("s")
    is_fast = cid == FAST_CID
    nch = jnp.where(is_fast, FAST_NCH, SLOW_NCH)

    @pl.when(is_fast)
    def _():
        pltpu.sync_copy(src3.at[pl.ds(sid * FAST_NCH, FAST_NCH)], src_v)
        pltpu.sync_copy(dst3.at[pl.ds(sid * FAST_NCH, FAST_NCH)], dst_v)

    is_tail = jnp.logical_and(jnp.logical_not(is_fast), sid == NS - 1)

    @pl.when(jnp.logical_and(jnp.logical_not(is_fast), sid < NS - 1))
    def _():
        pltpu.sync_copy(src3.at[pl.ds(FROWS + sid * SLOW_NCH, SLOW_NCH)],
                        src_v.at[pl.ds(0, SLOW_NCH)])
        pltpu.sync_copy(dst3.at[pl.ds(FROWS + sid * SLOW_NCH, SLOW_NCH)],
                        dst_v.at[pl.ds(0, SLOW_NCH)])

    @pl.when(is_tail)
    def _():
        base = FROWS + (NS - 1) * SLOW_NCH
        pltpu.sync_copy(src3.at[pl.ds(base, TAIL_REAL)],
                        src_v.at[pl.ds(0, TAIL_REAL)])
        pltpu.sync_copy(dst3.at[pl.ds(base, TAIL_REAL)],
                        dst_v.at[pl.ds(0, TAIL_REAL)])

        def padfill(i, _):
            r = TAIL_REAL + i // (CB // 16)
            c = (i % (CB // 16)) * 16
            src_v[r, pl.ds(c, 16)] = jnp.zeros((16,), jnp.int32)
            dst_v[r, pl.ds(c, 16)] = jnp.full((16,), PAD_DST, jnp.int32)
            return 0

        lax.fori_loop(0, TAIL_PAD * (CB // 16), padfill, 0)

    def zstore(i, _):
        rows_v[0, i // 4, pl.ds((i % 4) * 16, 16)] = jnp.zeros(
            (16,), jnp.float32)
        return 0

    lax.fori_loop(0, CB * 4, zstore, 0)
    for t in range(RPT // CB):
        pltpu.sync_copy(rows_v.at[0],
                        acc_s.at[pl.ds(sid * RPT + t * CB, CB)])
    plsc.subcore_barrier()

    pltpu.async_copy(m_hbm.at[src_v.at[0]], rows_v.at[0], gsem.at[0])
    pltpu.async_copy(m_hbm.at[src_v.at[1]], rows_v.at[1], gsem.at[1])

    def body(j, _):
        b = lax.rem(j, 4)
        pltpu.make_async_copy(m_hbm.at[src_v.at[j]], rows_v.at[b],
                              gsem.at[b]).wait()

        @pl.when(j >= 2)
        def _():
            b2 = lax.rem(j - 2, 4)
            pltpu.make_async_copy(rows_v.at[b2],
                                  acc_s.at[dst_v.at[j - 2]],
                                  ssem.at[b2]).wait()

        @pl.when(j + 2 < nch)
        def _():
            b3 = lax.rem(j + 2, 4)
            pltpu.async_copy(m_hbm.at[src_v.at[j + 2]], rows_v.at[b3],
                             gsem.at[b3])

        pltpu.async_copy(rows_v.at[b], acc_s.at[dst_v.at[j]],
                         ssem.at[b], add=True)
        return 0

    lax.fori_loop(0, nch, body, 0)
    for k in (2, 1):
        jj = nch - k
        b = lax.rem(jj, 4)
        pltpu.make_async_copy(rows_v.at[b], acc_s.at[dst_v.at[jj]],
                              ssem.at[b]).wait()
    plsc.subcore_barrier()
    pltpu.sync_copy(acc_s.at[pl.ds(sid * RPT, RPT)],
                    out_hbm.at[cid, pl.ds(sid * RPT, RPT)])


def _dense1a_body(x, w1, u_o):
    u_o[...] = jnp.dot(x[...], w1[...], preferred_element_type=jnp.float32)


def _dense1b_body(degp, u, lo_o, hi_o, dsc_o):
    deg = jnp.maximum(jnp.sum(degp[...], axis=0), 1.0)
    dsc = lax.rsqrt(deg)
    m = u[...] * dsc[:, None]
    lo_o[...] = m[:, :HF]
    hi_o[...] = m[:, HF:]
    dsc_o[...] = dsc


def _dense2h_body(a, dsc, w, m_o):
    h = jnp.maximum((a[0] + a[1]) * dsc[...][:, None], 0.0)
    m_o[...] = jnp.dot(h, w[...],
                       preferred_element_type=jnp.float32) * dsc[...][:, None]


def _dense3_body(alo, ahi, dsc, eps, wo, bo, z_o):
    t = jnp.concatenate([alo[0] + alo[1], ahi[0] + ahi[1]],
                        axis=1) * dsc[...][:, None]
    mu = jnp.concatenate([t[:, 0:LAT], t[:, 2 * LAT:3 * LAT]], axis=1)
    ls = jnp.concatenate([t[:, LAT:2 * LAT], t[:, 3 * LAT:4 * LAT]], axis=1)
    z = mu + eps[...] * jnp.exp(jnp.minimum(ls, MAXLS))
    z_o[...] = jnp.dot(z, wo[...],
                       preferred_element_type=jnp.float32) + bo[...]


_dense1a = pl.pallas_call(
    _dense1a_body,
    grid=(GRID,),
    in_specs=[
        pl.BlockSpec((BN, DF), lambda i: (i, 0)),
        pl.BlockSpec((DF, DF), lambda i: (0, 0)),
    ],
    out_specs=pl.BlockSpec((BN, DF), lambda i: (i, 0)),
    out_shape=jax.ShapeDtypeStruct((N, DF), jnp.float32),
)

_dense1b = pl.pallas_call(
    _dense1b_body,
    grid=(GRID,),
    in_specs=[
        pl.BlockSpec((NW, BN), lambda i: (0, i)),
        pl.BlockSpec((BN, DF), lambda i: (i, 0)),
    ],
    out_specs=[
        pl.BlockSpec((BN, HF), lambda i: (i, 0)),
        pl.BlockSpec((BN, HF), lambda i: (i, 0)),
        pl.BlockSpec((BN,), lambda i: (i,)),
    ],
    out_shape=[
        jax.ShapeDtypeStruct((N, HF), jnp.float32),
        jax.ShapeDtypeStruct((N, HF), jnp.float32),
        jax.ShapeDtypeStruct((N,), jnp.float32),
    ],
)

_agg_spec = pl.BlockSpec((NC, BN, HF), lambda i: (0, i, 0))

_dense2h = pl.pallas_call(
    _dense2h_body,
    grid=(GRID,),
    in_specs=[
        _agg_spec,
        pl.BlockSpec((BN,), lambda i: (i,)),
        pl.BlockSpec((HF, HF), lambda i: (0, 0)),
    ],
    out_specs=pl.BlockSpec((BN, HF), lambda i: (i, 0)),
    out_shape=jax.ShapeDtypeStruct((N, HF), jnp.float32),
)

_dense3 = pl.pallas_call(
    _dense3_body,
    grid=(GRID,),
    in_specs=[
        _agg_spec,
        _agg_spec,
        pl.BlockSpec((BN,), lambda i: (i,)),
        pl.BlockSpec((BN, 2 * LAT), lambda i: (i, 0)),
        pl.BlockSpec((2 * LAT, OUTD), lambda i: (0, 0)),
        pl.BlockSpec((1, OUTD), lambda i: (0, 0)),
    ],
    out_specs=pl.BlockSpec((BN, OUTD), lambda i: (i, 0)),
    out_shape=jax.ShapeDtypeStruct((N, OUTD), jnp.float32),
)


def kernel(x, W1_a, Wmu_a, Wls_a, W1_b, Wmu_b, Wls_b, W_out, b_out,
           edge_index):
    f32 = jnp.float32
    W1c = jnp.concatenate([W1_a, W1_b], axis=1)
    W2A = jnp.concatenate([Wmu_a, Wls_a], axis=1)
    W2B = jnp.concatenate([Wmu_b, Wls_b], axis=1)
    ke_a, ke_b = jax.random.split(jax.random.key(42), 2)

    src3 = edge_index[0].reshape(ECH, CB)
    dst3 = edge_index[1].reshape(ECH, CB)

    eps_a = jax.random.normal(ke_a, (N, LAT), dtype=f32)
    eps_b = jax.random.normal(ke_b, (N, LAT), dtype=f32)
    eps = jnp.concatenate([eps_a, eps_b], axis=1)

    degp = _sc_degree(edge_index[1])
    u = _dense1a(x, W1c)
    m1lo, m1hi, dsc = _dense1b(degp, u)
    a1lo = _sc_agg(m1lo, src3, dst3)
    m2lo = _dense2h(a1lo, dsc, W2A)
    a1hi = _sc_agg(m1hi, src3, dst3)
    m2hi = _dense2h(a1hi, dsc, W2B)
    a2lo = _sc_agg(m2lo, src3, dst3)
    a2hi = _sc_agg(m2hi, src3, dst3)
    z = _dense3(a2lo, a2hi, dsc, eps, W_out, b_out.reshape(1, OUTD))
    return z
